# baseline (speedup 1.0000x reference)
import functools

import jax
import jax.numpy as jnp
from jax import lax
from jax.experimental import pallas as pl
from jax.experimental.pallas import tpu as pltpu
from jax.experimental.pallas import tpu_sc as plsc

_HIDDEN = 32
_ROWS = 1024
_GROUPS = 32
_BLK = 128
_GPB = _BLK // _HIDDEN
_NCB = _GROUPS // _GPB
_NRS = 32 // _NCB
_RSEG = _ROWS // _NRS
_RCHUNK = 16
_NCHUNK = _RSEG // _RCHUNK
_NBUF = 6

_BLKQ = (0, 7, 15, 23)
_OFFQ = (0, 96, 64, 32)


@functools.partial(
    pl.kernel,
    mesh=plsc.VectorSubcoreMesh(core_axis_name="c", subcore_axis_name="s"),
    out_type=jax.ShapeDtypeStruct((_ROWS, _GROUPS * _HIDDEN), jnp.float32),
    scratch_types=[
        pltpu.VMEM((_NBUF, _GPB, _RCHUNK, _BLK), jnp.float32),
        pltpu.VMEM((_NBUF, _RCHUNK, _BLK), jnp.float32),
        pltpu.SemaphoreType.DMA,
        pltpu.SemaphoreType.DMA,
    ],
)
def _gather_kernel(in_hbm, out_hbm, inb_v, outb_v, insem, outsem):
    wid = lax.axis_index("s") * 2 + lax.axis_index("c")
    t = wid % _NCB
    rbase = (wid // _NCB) * _RSEG
    tcol = pl.multiple_of(t * _BLK, _BLK)
    z = jnp.zeros((16,), jnp.float32)

    def fire_in(c, slot):
        r0 = pl.multiple_of(rbase + c * _RCHUNK, _RCHUNK)
        cps = []
        for q in range(_GPB):
            col = pl.multiple_of((31 * t + _BLKQ[q]) * _BLK, _BLK)
            cps.append(pltpu.async_copy(
                in_hbm.at[pl.ds(r0, _RCHUNK), pl.ds(col, _BLK)],
                inb_v.at[slot, q], insem))
        return cps

    in_cps = [None] * _NBUF
    out_cps = [None] * _NBUF
    for p in range(_NBUF - 1):
        in_cps[p] = fire_in(p, p)
    for c in range(_NCHUNK):
        slot = c % _NBUF
        for cp in in_cps[slot]:
            cp.wait()
        if c + _NBUF - 1 < _NCHUNK:
            nxt = (c + _NBUF - 1) % _NBUF
            in_cps[nxt] = fire_in(c + _NBUF - 1, nxt)
        if out_cps[slot] is not None:
            out_cps[slot].wait()

        @plsc.parallel_loop(0, _RCHUNK, 1, unroll=2)
        def _row(i, slot=slot):
            for q in range(_GPB):
                off = _OFFQ[q]
                outb_v[slot, i, pl.ds(q * _HIDDEN, 16)] = \
                    inb_v[slot, q, i, pl.ds(off, 16)]
                outb_v[slot, i, pl.ds(q * _HIDDEN + 16, 16)] = \
                    inb_v[slot, q, i, pl.ds(off + 16, 16)]

        @pl.when(t == 0)
        def _zero(slot=slot):
            @plsc.parallel_loop(0, _RCHUNK, 1, unroll=2)
            def _zrow(i):
                outb_v[slot, i, pl.ds(0, 16)] = z
                outb_v[slot, i, pl.ds(16, 16)] = z

        r0 = pl.multiple_of(rbase + c * _RCHUNK, _RCHUNK)
        out_cps[slot] = pltpu.async_copy(
            outb_v.at[slot],
            out_hbm.at[pl.ds(r0, _RCHUNK), pl.ds(tcol, _BLK)], outsem)
    for cp in out_cps:
        if cp is not None:
            cp.wait()


def kernel(inputs):
    return _gather_kernel(inputs)


# speedup vs baseline: 1.0711x; 1.0711x over previous
import functools

import jax
import jax.numpy as jnp
from jax import lax
from jax.experimental import pallas as pl
from jax.experimental.pallas import tpu as pltpu
from jax.experimental.pallas import tpu_sc as plsc

_HIDDEN = 32
_ROWS = 1024
_GROUPS = 32
_BLK = 128
_GPB = _BLK // _HIDDEN
_NCB = _GROUPS // _GPB
_NRS = 32 // _NCB
_RSEG = _ROWS // _NRS
_RCHUNK = 32
_NCHUNK = _RSEG // _RCHUNK
_NBUF = 5

_BLKQ = (0, 7, 15, 23)
_OFFQ = (0, 96, 64, 32)


@functools.partial(
    pl.kernel,
    mesh=plsc.VectorSubcoreMesh(core_axis_name="c", subcore_axis_name="s"),
    out_type=jax.ShapeDtypeStruct((_ROWS, _GROUPS * _HIDDEN), jnp.float32),
    scratch_types=[
        pltpu.VMEM((_NBUF, _GPB, _RCHUNK, _BLK), jnp.float32),
        pltpu.VMEM((_NBUF, _RCHUNK, _BLK), jnp.float32),
        pltpu.SemaphoreType.DMA,
        pltpu.SemaphoreType.DMA,
    ],
)
def _gather_kernel(in_hbm, out_hbm, inb_v, outb_v, insem, outsem):
    wid = lax.axis_index("s") * 2 + lax.axis_index("c")
    t = wid % _NCB
    rbase = (wid // _NCB) * _RSEG
    tcol = pl.multiple_of(t * _BLK, _BLK)
    z = jnp.zeros((16,), jnp.float32)

    def fire_in(c, slot):
        r0 = pl.multiple_of(rbase + c * _RCHUNK, _RCHUNK)
        cps = []
        for q in range(_GPB):
            col = pl.multiple_of((31 * t + _BLKQ[q]) * _BLK, _BLK)
            cps.append(pltpu.async_copy(
                in_hbm.at[pl.ds(r0, _RCHUNK), pl.ds(col, _BLK)],
                inb_v.at[slot, q], insem))
        return cps

    in_cps = [None] * _NBUF
    out_cps = [None] * _NBUF
    for p in range(_NBUF - 1):
        in_cps[p] = fire_in(p, p)
    for c in range(_NCHUNK):
        slot = c % _NBUF
        for cp in in_cps[slot]:
            cp.wait()
        if c + _NBUF - 1 < _NCHUNK:
            nxt = (c + _NBUF - 1) % _NBUF
            in_cps[nxt] = fire_in(c + _NBUF - 1, nxt)
        if out_cps[slot] is not None:
            out_cps[slot].wait()

        @plsc.parallel_loop(0, _RCHUNK, 1, unroll=2)
        def _row(i, slot=slot):
            for q in range(_GPB):
                off = _OFFQ[q]
                outb_v[slot, i, pl.ds(q * _HIDDEN, 16)] = \
                    inb_v[slot, q, i, pl.ds(off, 16)]
                outb_v[slot, i, pl.ds(q * _HIDDEN + 16, 16)] = \
                    inb_v[slot, q, i, pl.ds(off + 16, 16)]

        @pl.when(t == 0)
        def _zero(slot=slot):
            @plsc.parallel_loop(0, _RCHUNK, 1, unroll=2)
            def _zrow(i):
                outb_v[slot, i, pl.ds(0, 16)] = z
                outb_v[slot, i, pl.ds(16, 16)] = z

        r0 = pl.multiple_of(rbase + c * _RCHUNK, _RCHUNK)
        out_cps[slot] = pltpu.async_copy(
            outb_v.at[slot],
            out_hbm.at[pl.ds(r0, _RCHUNK), pl.ds(tcol, _BLK)], outsem)
    for cp in out_cps:
        if cp is not None:
            cp.wait()


def kernel(inputs):
    return _gather_kernel(inputs)
